# baseline (speedup 1.0000x reference)
import functools

import jax
import jax.numpy as jnp
from jax import lax
from jax.experimental import pallas as pl
from jax.experimental.pallas import tpu as pltpu
from jax.experimental.pallas import tpu_sc as plsc

N_REAL = 10000
NP = 10240
NW = 32
ROWS = NP // NW
EB = 128
NB = 512
NGROUPS = 64

_CHUNK = {1024: (1024, 128), 512: (512, 128), 256: (256, 128), 516: (640, 128)}


def _dense(x, wk, wq, wv, ws, b3, dp, c):
    nc = dp // c
    nb = NP // NB
    din = x.shape[1]

    def body(x_ref, wk_ref, wq_ref, wv_ref, ws_ref, b_ref,
             ek_ref, eq_ref, v_ref, s_ref):
        xb = x_ref[...]
        ek_ref[0] = jnp.exp(-jnp.dot(xb, wk_ref[...],
                                     preferred_element_type=jnp.float32))
        eq_ref[0] = jnp.exp(-jnp.dot(xb, wq_ref[...],
                                     preferred_element_type=jnp.float32))
        v_ref[0] = jnp.dot(xb, wv_ref[...], preferred_element_type=jnp.float32)
        s_ref[...] = jnp.dot(xb, ws_ref[...],
                             preferred_element_type=jnp.float32) + b_ref[0]

    chunk3 = jax.ShapeDtypeStruct((nc, NP, c), jnp.float32)
    return pl.pallas_call(
        body,
        grid=(nc, nb),
        in_specs=[
            pl.BlockSpec((NB, din), lambda ci, bi: (bi, 0)),
            pl.BlockSpec((din, c), lambda ci, bi: (0, ci)),
            pl.BlockSpec((din, c), lambda ci, bi: (0, ci)),
            pl.BlockSpec((din, c), lambda ci, bi: (0, ci)),
            pl.BlockSpec((din, c), lambda ci, bi: (0, ci)),
            pl.BlockSpec((1, 1, c), lambda ci, bi: (ci, 0, 0)),
        ],
        out_specs=[
            pl.BlockSpec((1, NB, c), lambda ci, bi: (ci, bi, 0)),
            pl.BlockSpec((1, NB, c), lambda ci, bi: (ci, bi, 0)),
            pl.BlockSpec((1, NB, c), lambda ci, bi: (ci, bi, 0)),
            pl.BlockSpec((NB, c), lambda ci, bi: (bi, ci)),
        ],
        out_shape=[chunk3, chunk3, chunk3,
                   jax.ShapeDtypeStruct((NP, dp), jnp.float32)],
    )(x, wk, wq, wv, ws, b3)


def _edge(ek3, eq3, v3, meta, src_s, dst_s, dp, c):
    nc = dp // c
    ng = c // 16
    mesh = plsc.VectorSubcoreMesh(core_axis_name="c", subcore_axis_name="s",
                                  num_cores=2, num_subcores=16)

    @functools.partial(
        pl.kernel,
        out_type=jax.ShapeDtypeStruct((nc, NP, c), jnp.float32),
        mesh=mesh,
        scratch_types=[
            pltpu.VMEM((ROWS, c), jnp.float32),
            pltpu.VMEM((ROWS, c), jnp.float32),
            pltpu.VMEM((EB,), jnp.int32),
            pltpu.VMEM((EB + 16,), jnp.int32),
            pltpu.VMEM((EB, c), jnp.float32),
            pltpu.VMEM((EB, c), jnp.float32),
            pltpu.VMEM((80,), jnp.int32),
            pltpu.SemaphoreType.DMA,
            pltpu.SemaphoreType.DMA,
        ],
    )
    def k(ek_hbm, eq_hbm, v_hbm, meta_hbm, src_hbm, dst_hbm, agg_hbm,
          agg_v, ek_v, si_v, di_v, eq_v, vv_v, meta_v, sem0, sem1):
        wid = lax.axis_index("s") * 2 + lax.axis_index("c")
        base = wid * ROWS
        pltpu.sync_copy(meta_hbm, meta_v)
        e_lo = meta_v[pl.ds(wid, 16)][0]
        e_hi = meta_v[pl.ds(wid + NW, 16)][0]
        e0 = (e_lo // 8) * 8
        nblk = (e_hi - e0 + EB - 1) // EB

        for ci in range(nc):
            def zrow(r, carry):
                for g in range(ng):
                    agg_v[r, pl.ds(g * 16, 16)] = jnp.zeros((16,), jnp.float32)
                return carry
            lax.fori_loop(0, ROWS, zrow, 0)

            pltpu.sync_copy(ek_hbm.at[ci, pl.ds(base, ROWS)], ek_v)

            def blk(ib, carry):
                bs = e0 + ib * EB
                pltpu.sync_copy(src_hbm.at[pl.ds(bs, EB)], si_v)
                pltpu.sync_copy(dst_hbm.at[pl.ds(bs, EB + 16)], di_v)
                cp0 = pltpu.async_copy(eq_hbm.at[ci].at[si_v], eq_v, sem0)
                cp1 = pltpu.async_copy(v_hbm.at[ci].at[si_v], vv_v, sem1)
                cp0.wait()
                cp1.wait()
                lo = jnp.maximum(e_lo, bs) - bs
                hi = jnp.minimum(e_hi, bs + EB) - bs

                def edge(j, carry2):
                    r = di_v[pl.ds(j, 16)][0] - base
                    for g in range(ng):
                        sl = pl.ds(g * 16, 16)
                        t = ek_v[r, sl] * eq_v[j, sl]
                        contrib = vv_v[j, sl] / (t + 1.0)
                        plsc.addupdate(agg_v.at[r, sl], contrib)
                    return carry2

                lax.fori_loop(lo, hi, edge, 0)
                return carry

            lax.fori_loop(0, nblk, blk, 0)
            pltpu.sync_copy(agg_v, agg_hbm.at[ci, pl.ds(base, ROWS)])

    return k(ek3, eq3, v3, meta, src_s, dst_s)


def _post(agg3, s2, g3, bt3, dp, c):
    nc = dp // c
    nb = NP // NB
    inv_n = 1.0 / N_REAL

    def body(agg_ref, s_ref, g_ref, bt_ref, out_ref, sum_ref, ssq_ref):
        p = pl.program_id(0)
        ci = pl.program_id(1)
        bi = pl.program_id(2)
        row = bi * NB + lax.broadcasted_iota(jnp.int32, (NB, 1), 0)
        live = row < N_REAL
        y = jnp.where(live, jnp.maximum(agg_ref[0] + s_ref[...], 0.0), 0.0)

        @pl.when(p == 0)
        def _stats():
            cs = jnp.sum(y, axis=0, keepdims=True)
            cq = jnp.sum(y * y, axis=0, keepdims=True)

            @pl.when(bi == 0)
            def _init():
                sum_ref[ci] = cs
                ssq_ref[ci] = cq

            @pl.when(bi != 0)
            def _acc():
                sum_ref[ci] += cs
                ssq_ref[ci] += cq

        @pl.when(p == 1)
        def _norm():
            mu = sum_ref[ci] * inv_n
            var = ssq_ref[ci] * inv_n - mu * mu
            rs = lax.rsqrt(var + 1e-5)
            out_ref[...] = jnp.where(
                live, (y - mu) * (rs * g_ref[0]) + bt_ref[0], 0.0)

    return pl.pallas_call(
        body,
        grid=(2, nc, nb),
        in_specs=[
            pl.BlockSpec((1, NB, c), lambda p, ci, bi: (ci, bi, 0)),
            pl.BlockSpec((NB, c), lambda p, ci, bi: (bi, ci)),
            pl.BlockSpec((1, 1, c), lambda p, ci, bi: (ci, 0, 0)),
            pl.BlockSpec((1, 1, c), lambda p, ci, bi: (ci, 0, 0)),
        ],
        out_specs=pl.BlockSpec((NB, c), lambda p, ci, bi: (bi, ci)),
        out_shape=jax.ShapeDtypeStruct((NP, dp), jnp.float32),
        scratch_shapes=[
            pltpu.VMEM((nc, 1, c), jnp.float32),
            pltpu.VMEM((nc, 1, c), jnp.float32),
        ],
    )(agg3, s2, g3, bt3)


def _pool(x5, batch3, dp):
    nb = NP // NB

    def body(x_ref, b_ref, out_ref, acc_ref):
        bi = pl.program_id(0)
        ids = b_ref[0]
        iot = lax.broadcasted_iota(jnp.int32, (NGROUPS, NB), 0)
        oh = (ids == iot).astype(jnp.float32)
        contrib = jnp.dot(oh, x_ref[...], preferred_element_type=jnp.float32)

        @pl.when(bi == 0)
        def _init():
            acc_ref[...] = contrib

        @pl.when(bi != 0)
        def _acc():
            acc_ref[...] += contrib

        @pl.when(bi == nb - 1)
        def _out():
            out_ref[...] = acc_ref[...]

    return pl.pallas_call(
        body,
        grid=(nb,),
        in_specs=[
            pl.BlockSpec((NB, dp), lambda bi: (bi, 0)),
            pl.BlockSpec((1, 1, NB), lambda bi: (bi, 0, 0)),
        ],
        out_specs=pl.BlockSpec((NGROUPS, dp), lambda bi: (0, 0)),
        out_shape=jax.ShapeDtypeStruct((NGROUPS, dp), jnp.float32),
        scratch_shapes=[pltpu.VMEM((NGROUPS, dp), jnp.float32)],
    )(x5, batch3)


def kernel(x, edge_index, batch, params):
    src = edge_index[0].astype(jnp.int32)
    dst = edge_index[1].astype(jnp.int32)
    e = src.shape[0]

    order = jnp.argsort(dst)
    src_s = src[order]
    dst_s = dst[order]
    bases = jnp.arange(NW, dtype=jnp.int32) * ROWS
    starts = jnp.searchsorted(dst_s, bases).astype(jnp.int32)
    ends = jnp.searchsorted(dst_s, bases + ROWS).astype(jnp.int32)
    meta = jnp.concatenate([starts, ends, jnp.zeros((16,), jnp.int32)])
    pad_e = jnp.zeros((2 * EB,), jnp.int32)
    src_p = jnp.concatenate([src_s, pad_e])
    dst_p = jnp.concatenate([dst_s, pad_e])

    x_cur = jnp.pad(x, ((0, NP - N_REAL), (0, 0)))
    batch_p = jnp.pad(batch.astype(jnp.int32), (0, NP - N_REAL),
                      constant_values=NGROUPS)
    batch3 = batch_p.reshape(NP // NB, 1, NB)

    dp_prev = x.shape[1]
    dp = dp_prev
    for (wk, wq, wv, ws, b, gamma, beta) in params:
        din, dout = wk.shape
        dp, c = _CHUNK[dout]
        nc = dp // c
        pad_w = ((0, dp_prev - din), (0, dp - dout))
        wk_p = jnp.pad(wk, pad_w)
        wq_p = jnp.pad(wq, pad_w)
        wv_p = jnp.pad(wv, pad_w)
        ws_p = jnp.pad(ws, pad_w)
        b3 = jnp.pad(b, (0, dp - dout)).reshape(nc, 1, c)
        g3 = jnp.pad(gamma, (0, dp - dout)).reshape(nc, 1, c)
        bt3 = jnp.pad(beta, (0, dp - dout)).reshape(nc, 1, c)

        ek3, eq3, v3, s2 = _dense(x_cur, wk_p, wq_p, wv_p, ws_p, b3, dp, c)
        agg3 = _edge(ek3, eq3, v3, meta, src_p, dst_p, dp, c)
        x_cur = _post(agg3, s2, g3, bt3, dp, c)
        dp_prev = dp

    return _pool(x_cur, batch3, dp)


# speedup vs baseline: 5.2065x; 5.2065x over previous
import functools

import jax
import jax.numpy as jnp
from jax import lax
from jax.experimental import pallas as pl
from jax.experimental.pallas import tpu as pltpu
from jax.experimental.pallas import tpu_sc as plsc

N_REAL = 10000
NP = 10240
NW = 32
ROWS = NP // NW
HROWS = ROWS // 2
EB = 128
NB = 512
NGROUPS = 64

_CHUNK = {1024: (1024, 128), 512: (512, 128), 256: (256, 128), 516: (640, 128)}


def _dense(x, wk, wq, wv, ws, b3, dp, c):
    nc = dp // c
    nb = NP // NB
    din = x.shape[1]

    def body(x_ref, wk_ref, wq_ref, wv_ref, ws_ref, b_ref,
             ek_ref, eq_ref, v_ref, s_ref):
        xb = x_ref[...]

        def mm(w_ref):
            return jnp.dot(xb, w_ref[...], preferred_element_type=jnp.float32)

        ek_ref[0] = jnp.exp(-mm(wk_ref))
        eq_ref[0] = jnp.exp(-mm(wq_ref))
        v_ref[0] = mm(wv_ref)
        s_ref[...] = mm(ws_ref) + b_ref[0]

    chunk3 = jax.ShapeDtypeStruct((nc, NP, c), jnp.float32)
    return pl.pallas_call(
        body,
        grid=(nc, nb),
        in_specs=[
            pl.BlockSpec((NB, din), lambda ci, bi: (bi, 0)),
            pl.BlockSpec((din, c), lambda ci, bi: (0, ci)),
            pl.BlockSpec((din, c), lambda ci, bi: (0, ci)),
            pl.BlockSpec((din, c), lambda ci, bi: (0, ci)),
            pl.BlockSpec((din, c), lambda ci, bi: (0, ci)),
            pl.BlockSpec((1, 1, c), lambda ci, bi: (ci, 0, 0)),
        ],
        out_specs=[
            pl.BlockSpec((1, NB, c), lambda ci, bi: (ci, bi, 0)),
            pl.BlockSpec((1, NB, c), lambda ci, bi: (ci, bi, 0)),
            pl.BlockSpec((1, NB, c), lambda ci, bi: (ci, bi, 0)),
            pl.BlockSpec((NB, c), lambda ci, bi: (bi, ci)),
        ],
        out_shape=[chunk3, chunk3, chunk3,
                   jax.ShapeDtypeStruct((NP, dp), jnp.float32)],
    )(x, wk, wq, wv, ws, b3)


def _edge(ek3, eq3, v3, rowptr, src_s, dst_s, dp, c):
    nc = dp // c
    ng = c // 16
    mesh = plsc.VectorSubcoreMesh(core_axis_name="c", subcore_axis_name="s",
                                  num_cores=2, num_subcores=16)

    @functools.partial(
        pl.kernel,
        out_type=jax.ShapeDtypeStruct((nc, NP, c), jnp.float32),
        mesh=mesh,
        scratch_types=[
            pltpu.VMEM((HROWS, c), jnp.float32),
            pltpu.VMEM((HROWS, c), jnp.float32),
            pltpu.VMEM((EB,), jnp.int32),
            pltpu.VMEM((EB,), jnp.int32),
            pltpu.VMEM((EB + 16,), jnp.int32),
            pltpu.VMEM((EB + 16,), jnp.int32),
            pltpu.VMEM((EB, c), jnp.float32),
            pltpu.VMEM((EB, c), jnp.float32),
            pltpu.VMEM((EB, c), jnp.float32),
            pltpu.VMEM((EB, c), jnp.float32),
            pltpu.VMEM((HROWS + 32,), jnp.int32),
            pltpu.SMEM((8,), jnp.int32),
            pltpu.SemaphoreType.DMA,
            pltpu.SemaphoreType.DMA,
            pltpu.SemaphoreType.DMA,
            pltpu.SemaphoreType.DMA,
        ],
    )
    def k(ek_hbm, eq_hbm, v_hbm, rp_hbm, src_hbm, dst_hbm, agg_hbm,
          agg_v, ek_v, si0, si1, di0, di1, eq0, eq1, vv0, vv1, rp_v, rs_v,
          sa0, sa1, sb0, sb1):
        sems = ((sa0, sb0), (sa1, sb1))
        sis = (si0, si1)
        dis = (di0, di1)
        eqs = (eq0, eq1)
        vvs = (vv0, vv1)
        wid = lax.axis_index("s") * 2 + lax.axis_index("c")
        sls = [pl.ds(g2 * 16, 16) for g2 in range(ng)]

        def rd(idx):
            return rp_v[pl.ds(idx, 16)][0]

        for ci in range(nc):
            for h in (0, 1):
                base = wid * ROWS + h * HROWS
                pltpu.sync_copy(rp_hbm.at[pl.ds(base, HROWS + 32)], rp_v)
                e_lo = rd(0)
                e_hi = rd(HROWS)
                e0 = (e_lo // 8) * 8
                nblk = (e_hi - e0 + EB - 1) // EB

                def zrow(r, carry):
                    for g in range(ng):
                        agg_v[r, pl.ds(g * 16, 16)] = jnp.zeros((16,),
                                                                jnp.float32)
                    return carry
                lax.fori_loop(0, HROWS, zrow, 0)

                pltpu.sync_copy(ek_hbm.at[ci, pl.ds(base, HROWS)], ek_v)

                def issue(g, buf):
                    bs = e0 + g * EB
                    pltpu.sync_copy(src_hbm.at[pl.ds(bs, EB)], sis[buf])
                    pltpu.sync_copy(dst_hbm.at[pl.ds(bs, EB + 16)], dis[buf])
                    pltpu.async_copy(eq_hbm.at[ci].at[sis[buf]],
                                     eqs[buf], sems[buf][0])
                    pltpu.async_copy(v_hbm.at[ci].at[sis[buf]],
                                     vvs[buf], sems[buf][1])

                def compute(g, buf):
                    bs = e0 + g * EB
                    pltpu.make_async_copy(eq_hbm.at[ci].at[sis[buf]],
                                          eqs[buf], sems[buf][0]).wait()
                    pltpu.make_async_copy(v_hbm.at[ci].at[sis[buf]],
                                          vvs[buf], sems[buf][1]).wait()
                    lo = jnp.maximum(e_lo, bs) - bs
                    hi = jnp.minimum(e_hi, bs + EB) - bs
                    rstart = rs_v[0]
                    jsafe = jnp.maximum(hi - 1, 0)
                    rlast = dis[buf][pl.ds(jsafe, 16)][0] - base
                    nrows = jnp.where(hi > lo, rlast - rstart + 1, 0)

                    def row(ri, jlo):
                        r = rstart + ri
                        rend = rd(r + 1) - bs
                        jhi = jnp.maximum(jnp.minimum(rend, hi), jlo)
                        ekl = [ek_v[r, sl] for sl in sls]

                        def erun(jj, accs):
                            eql = [eqs[buf][jj, sl] for sl in sls]
                            vvl = [vvs[buf][jj, sl] for sl in sls]
                            tl = [a * b for a, b in zip(ekl, eql)]
                            dl = [t + 1.0 for t in tl]
                            cl = [v / d for v, d in zip(vvl, dl)]
                            return tuple(a + cv for a, cv in
                                         zip(accs, cl))

                        accs = lax.fori_loop(
                            jlo, jhi, erun,
                            tuple(jnp.zeros((16,), jnp.float32)
                                  for _ in range(ng)))

                        for sl, av in zip(sls, accs):
                            plsc.addupdate(agg_v.at[r, sl], av)

                        return jhi

                    lax.fori_loop(0, nrows, row, lo)
                    rs_v[0] = jnp.where(hi > lo, rlast, rstart)

                rs_v[0] = 0

                @pl.when(nblk > 0)
                def _prime():
                    issue(0, 0)

                def pair(ip, carry):
                    for b in (0, 1):
                        g = ip * 2 + b

                        @pl.when(g < nblk)
                        def _step():
                            @pl.when(g + 1 < nblk)
                            def _ahead():
                                issue(g + 1, 1 - b)
                            compute(g, b)
                    return carry

                lax.fori_loop(0, (nblk + 1) // 2, pair, 0)
                pltpu.sync_copy(agg_v, agg_hbm.at[ci, pl.ds(base, HROWS)])

    return k(ek3, eq3, v3, rowptr, src_s, dst_s)


def _post(agg3, s2, g3, bt3, dp, c):
    nc = dp // c
    nb = NP // NB
    inv_n = 1.0 / N_REAL

    def body(agg_ref, s_ref, g_ref, bt_ref, out_ref, sum_ref, ssq_ref):
        p = pl.program_id(0)
        ci = pl.program_id(1)
        bi = pl.program_id(2)
        row = bi * NB + lax.broadcasted_iota(jnp.int32, (NB, 1), 0)
        live = row < N_REAL
        y = jnp.where(live, jnp.maximum(agg_ref[0] + s_ref[...], 0.0), 0.0)

        @pl.when(p == 0)
        def _stats():
            cs = jnp.sum(y, axis=0, keepdims=True)
            cq = jnp.sum(y * y, axis=0, keepdims=True)

            @pl.when(bi == 0)
            def _init():
                sum_ref[ci] = cs
                ssq_ref[ci] = cq

            @pl.when(bi != 0)
            def _acc():
                sum_ref[ci] += cs
                ssq_ref[ci] += cq

        @pl.when(p == 1)
        def _norm():
            mu = sum_ref[ci] * inv_n
            var = ssq_ref[ci] * inv_n - mu * mu
            rs = lax.rsqrt(var + 1e-5)
            out_ref[...] = jnp.where(
                live, (y - mu) * (rs * g_ref[0]) + bt_ref[0], 0.0)

    return pl.pallas_call(
        body,
        grid=(2, nc, nb),
        in_specs=[
            pl.BlockSpec((1, NB, c), lambda p, ci, bi: (ci, bi, 0)),
            pl.BlockSpec((NB, c), lambda p, ci, bi: (bi, ci)),
            pl.BlockSpec((1, 1, c), lambda p, ci, bi: (ci, 0, 0)),
            pl.BlockSpec((1, 1, c), lambda p, ci, bi: (ci, 0, 0)),
        ],
        out_specs=pl.BlockSpec((NB, c), lambda p, ci, bi: (bi, ci)),
        out_shape=jax.ShapeDtypeStruct((NP, dp), jnp.float32),
        scratch_shapes=[
            pltpu.VMEM((nc, 1, c), jnp.float32),
            pltpu.VMEM((nc, 1, c), jnp.float32),
        ],
    )(agg3, s2, g3, bt3)


def _pool(x5, batch3, dp):
    nb = NP // NB

    def body(x_ref, b_ref, out_ref, acc_ref):
        bi = pl.program_id(0)
        ids = b_ref[0]
        iot = lax.broadcasted_iota(jnp.int32, (NGROUPS, NB), 0)
        oh = (ids == iot).astype(jnp.float32)
        contrib = jnp.dot(oh, x_ref[...], preferred_element_type=jnp.float32,
                          precision=lax.Precision.HIGHEST)

        @pl.when(bi == 0)
        def _init():
            acc_ref[...] = contrib

        @pl.when(bi != 0)
        def _acc():
            acc_ref[...] += contrib

        @pl.when(bi == nb - 1)
        def _out():
            out_ref[...] = acc_ref[...]

    return pl.pallas_call(
        body,
        grid=(nb,),
        in_specs=[
            pl.BlockSpec((NB, dp), lambda bi: (bi, 0)),
            pl.BlockSpec((1, 1, NB), lambda bi: (bi, 0, 0)),
        ],
        out_specs=pl.BlockSpec((NGROUPS, dp), lambda bi: (0, 0)),
        out_shape=jax.ShapeDtypeStruct((NGROUPS, dp), jnp.float32),
        scratch_shapes=[pltpu.VMEM((NGROUPS, dp), jnp.float32)],
    )(x5, batch3)


def kernel(x, edge_index, batch, params):
    src = edge_index[0].astype(jnp.int32)
    dst = edge_index[1].astype(jnp.int32)
    e = src.shape[0]

    order = jnp.argsort(dst)
    src_s = src[order]
    dst_s = dst[order]
    rowptr = jnp.searchsorted(
        dst_s, jnp.arange(NP + 32, dtype=jnp.int32)).astype(jnp.int32)
    pad_e = jnp.zeros((2 * EB,), jnp.int32)
    src_p = jnp.concatenate([src_s, pad_e])
    dst_p = jnp.concatenate([dst_s, pad_e])

    x_cur = jnp.pad(x, ((0, NP - N_REAL), (0, 0)))
    batch_p = jnp.pad(batch.astype(jnp.int32), (0, NP - N_REAL),
                      constant_values=NGROUPS)
    batch3 = batch_p.reshape(NP // NB, 1, NB)

    dp_prev = x.shape[1]
    dp = dp_prev
    for (wk, wq, wv, ws, b, gamma, beta) in params:
        din, dout = wk.shape
        dp, c = _CHUNK[dout]
        nc = dp // c
        pad_w = ((0, dp_prev - din), (0, dp - dout))
        wk_p = jnp.pad(wk, pad_w)
        wq_p = jnp.pad(wq, pad_w)
        wv_p = jnp.pad(wv, pad_w)
        ws_p = jnp.pad(ws, pad_w)
        b3 = jnp.pad(b, (0, dp - dout)).reshape(nc, 1, c)
        g3 = jnp.pad(gamma, (0, dp - dout)).reshape(nc, 1, c)
        bt3 = jnp.pad(beta, (0, dp - dout)).reshape(nc, 1, c)

        ek3, eq3, v3, s2 = _dense(x_cur, wk_p, wq_p, wv_p, ws_p, b3, dp, c)
        agg3 = _edge(ek3, eq3, v3, rowptr, src_p, dst_p, dp, c)
        x_cur = _post(agg3, s2, g3, bt3, dp, c)
        dp_prev = dp

    return _pool(x_cur, batch3, dp)


# speedup vs baseline: 5.7868x; 1.1115x over previous
import functools

import jax
import jax.numpy as jnp
from jax import lax
from jax.experimental import pallas as pl
from jax.experimental.pallas import tpu as pltpu
from jax.experimental.pallas import tpu_sc as plsc

N_REAL = 10000
NP = 10240
NW = 32
ROWS = NP // NW
HROWS = ROWS // 2
EB = 128
NB = 512
NGROUPS = 64

_CHUNK = {1024: (1024, 128), 512: (512, 128), 256: (256, 128), 516: (640, 128)}


def _dense(x, wk, wq, wv, ws, b3, dp, c):
    nc = dp // c
    nb = NP // NB
    din = x.shape[1]

    def body(x_ref, wk_ref, wq_ref, wv_ref, ws_ref, b_ref,
             ek_ref, eq_ref, v_ref, s_ref):
        xb = x_ref[...]

        def mm(w_ref):
            return jnp.dot(xb, w_ref[...], preferred_element_type=jnp.float32)

        ek_ref[0] = jnp.exp(-mm(wk_ref))
        eq_ref[0] = jnp.exp(-mm(wq_ref))
        v_ref[0] = mm(wv_ref)
        s_ref[...] = mm(ws_ref) + b_ref[0]

    chunk3 = jax.ShapeDtypeStruct((nc, NP, c), jnp.float32)
    return pl.pallas_call(
        body,
        grid=(nc, nb),
        in_specs=[
            pl.BlockSpec((NB, din), lambda ci, bi: (bi, 0)),
            pl.BlockSpec((din, c), lambda ci, bi: (0, ci)),
            pl.BlockSpec((din, c), lambda ci, bi: (0, ci)),
            pl.BlockSpec((din, c), lambda ci, bi: (0, ci)),
            pl.BlockSpec((din, c), lambda ci, bi: (0, ci)),
            pl.BlockSpec((1, 1, c), lambda ci, bi: (ci, 0, 0)),
        ],
        out_specs=[
            pl.BlockSpec((1, NB, c), lambda ci, bi: (ci, bi, 0)),
            pl.BlockSpec((1, NB, c), lambda ci, bi: (ci, bi, 0)),
            pl.BlockSpec((1, NB, c), lambda ci, bi: (ci, bi, 0)),
            pl.BlockSpec((NB, c), lambda ci, bi: (bi, ci)),
        ],
        out_shape=[chunk3, chunk3, chunk3,
                   jax.ShapeDtypeStruct((NP, dp), jnp.float32)],
    )(x, wk, wq, wv, ws, b3)


def _edge(ek3, eq3, v3, rowptr, src_s, dst_s, dp, c):
    nc = dp // c
    ng = c // 16
    mesh = plsc.VectorSubcoreMesh(core_axis_name="c", subcore_axis_name="s",
                                  num_cores=2, num_subcores=16)

    @functools.partial(
        pl.kernel,
        out_type=jax.ShapeDtypeStruct((nc, NP, c), jnp.float32),
        mesh=mesh,
        scratch_types=[
            pltpu.VMEM((HROWS, c), jnp.float32),
            pltpu.VMEM((HROWS, c), jnp.float32),
            pltpu.VMEM((EB,), jnp.int32),
            pltpu.VMEM((EB,), jnp.int32),
            pltpu.VMEM((EB + 16,), jnp.int32),
            pltpu.VMEM((EB + 16,), jnp.int32),
            pltpu.VMEM((EB, c), jnp.float32),
            pltpu.VMEM((EB, c), jnp.float32),
            pltpu.VMEM((EB, c), jnp.float32),
            pltpu.VMEM((EB, c), jnp.float32),
            pltpu.VMEM((HROWS + 32,), jnp.int32),
            pltpu.SMEM((8,), jnp.int32),
            pltpu.SemaphoreType.DMA,
            pltpu.SemaphoreType.DMA,
            pltpu.SemaphoreType.DMA,
            pltpu.SemaphoreType.DMA,
            pltpu.SemaphoreType.DMA,
            pltpu.SemaphoreType.DMA,
        ],
    )
    def k(ek_hbm, eq_hbm, v_hbm, rp_hbm, src_hbm, dst_hbm, agg_hbm,
          agg_v, ek_v, si0, si1, di0, di1, eq0, eq1, vv0, vv1, rp_v, rs_v,
          sa0, sa1, sb0, sb1, sx0, sx1):
        sems = ((sa0, sb0), (sa1, sb1))
        isems = (sx0, sx1)
        sis = (si0, si1)
        dis = (di0, di1)
        eqs = (eq0, eq1)
        vvs = (vv0, vv1)
        wid = lax.axis_index("s") * 2 + lax.axis_index("c")
        sls = [pl.ds(g2 * 16, 16) for g2 in range(ng)]

        def rd(idx):
            return rp_v[pl.ds(idx, 16)][0]

        for ci in range(nc):
            for h in (0, 1):
                base = wid * ROWS + h * HROWS
                pltpu.sync_copy(rp_hbm.at[pl.ds(base, HROWS + 32)], rp_v)
                e_lo = rd(0)
                e_hi = rd(HROWS)
                e0 = (e_lo // 8) * 8
                nblk = (e_hi - e0 + EB - 1) // EB

                def zrow(r, carry):
                    for g in range(ng):
                        agg_v[r, pl.ds(g * 16, 16)] = jnp.zeros((16,),
                                                                jnp.float32)
                    return carry
                lax.fori_loop(0, HROWS, zrow, 0)

                pltpu.sync_copy(ek_hbm.at[ci, pl.ds(base, HROWS)], ek_v)

                def issue_idx(g, buf):
                    bs = e0 + g * EB
                    pltpu.async_copy(src_hbm.at[pl.ds(bs, EB)], sis[buf],
                                     isems[buf])
                    pltpu.async_copy(dst_hbm.at[pl.ds(bs, EB + 16)],
                                     dis[buf], isems[buf])

                def issue_gather(g, buf):
                    bs = e0 + g * EB
                    pltpu.make_async_copy(src_hbm.at[pl.ds(bs, EB)],
                                          sis[buf], isems[buf]).wait()
                    pltpu.make_async_copy(dst_hbm.at[pl.ds(bs, EB + 16)],
                                          dis[buf], isems[buf]).wait()
                    pltpu.async_copy(eq_hbm.at[ci].at[sis[buf]],
                                     eqs[buf], sems[buf][0])
                    pltpu.async_copy(v_hbm.at[ci].at[sis[buf]],
                                     vvs[buf], sems[buf][1])

                def compute(g, buf):
                    bs = e0 + g * EB
                    pltpu.make_async_copy(eq_hbm.at[ci].at[sis[buf]],
                                          eqs[buf], sems[buf][0]).wait()
                    pltpu.make_async_copy(v_hbm.at[ci].at[sis[buf]],
                                          vvs[buf], sems[buf][1]).wait()
                    lo = jnp.maximum(e_lo, bs) - bs
                    hi = jnp.minimum(e_hi, bs + EB) - bs
                    rstart = rs_v[0]
                    jsafe = jnp.maximum(hi - 1, 0)
                    rlast = dis[buf][pl.ds(jsafe, 16)][0] - base
                    nrows = jnp.where(hi > lo, rlast - rstart + 1, 0)

                    def row(ri, jlo):
                        r = rstart + ri
                        rend = rd(r + 1) - bs
                        jhi = jnp.maximum(jnp.minimum(rend, hi), jlo)
                        ekl = [ek_v[r, sl] for sl in sls]

                        def erun(jj, accs):
                            eql = [eqs[buf][jj, sl] for sl in sls]
                            vvl = [vvs[buf][jj, sl] for sl in sls]
                            tl = [a * b for a, b in zip(ekl, eql)]
                            dl = [t + 1.0 for t in tl]
                            cl = [v / d for v, d in zip(vvl, dl)]
                            return tuple(a + cv for a, cv in
                                         zip(accs, cl))

                        accs = lax.fori_loop(
                            jlo, jhi, erun,
                            tuple(jnp.zeros((16,), jnp.float32)
                                  for _ in range(ng)))

                        for sl, av in zip(sls, accs):
                            plsc.addupdate(agg_v.at[r, sl], av)

                        return jhi

                    lax.fori_loop(0, nrows, row, lo)
                    rs_v[0] = jnp.where(hi > lo, rlast, rstart)

                rs_v[0] = 0

                @pl.when(nblk > 0)
                def _prime0():
                    issue_idx(0, 0)

                @pl.when(nblk > 1)
                def _prime1():
                    issue_idx(1, 1)

                @pl.when(nblk > 0)
                def _prime2():
                    issue_gather(0, 0)

                def pair(ip, carry):
                    for b in (0, 1):
                        g = ip * 2 + b

                        @pl.when(g < nblk)
                        def _step():
                            @pl.when(g + 1 < nblk)
                            def _ahead():
                                issue_gather(g + 1, 1 - b)
                            compute(g, b)

                            @pl.when(g + 2 < nblk)
                            def _stage():
                                issue_idx(g + 2, b)
                    return carry

                lax.fori_loop(0, (nblk + 1) // 2, pair, 0)
                pltpu.sync_copy(agg_v, agg_hbm.at[ci, pl.ds(base, HROWS)])

    return k(ek3, eq3, v3, rowptr, src_s, dst_s)


def _post(agg3, s2, g3, bt3, dp, c):
    nc = dp // c
    nb = NP // NB
    inv_n = 1.0 / N_REAL

    def body(agg_ref, s_ref, g_ref, bt_ref, out_ref, sum_ref, ssq_ref):
        p = pl.program_id(0)
        ci = pl.program_id(1)
        bi = pl.program_id(2)
        row = bi * NB + lax.broadcasted_iota(jnp.int32, (NB, 1), 0)
        live = row < N_REAL
        y = jnp.where(live, jnp.maximum(agg_ref[0] + s_ref[...], 0.0), 0.0)

        @pl.when(p == 0)
        def _stats():
            cs = jnp.sum(y, axis=0, keepdims=True)
            cq = jnp.sum(y * y, axis=0, keepdims=True)

            @pl.when(bi == 0)
            def _init():
                sum_ref[ci] = cs
                ssq_ref[ci] = cq

            @pl.when(bi != 0)
            def _acc():
                sum_ref[ci] += cs
                ssq_ref[ci] += cq

        @pl.when(p == 1)
        def _norm():
            mu = sum_ref[ci] * inv_n
            var = ssq_ref[ci] * inv_n - mu * mu
            rs = lax.rsqrt(var + 1e-5)
            out_ref[...] = jnp.where(
                live, (y - mu) * (rs * g_ref[0]) + bt_ref[0], 0.0)

    return pl.pallas_call(
        body,
        grid=(2, nc, nb),
        in_specs=[
            pl.BlockSpec((1, NB, c), lambda p, ci, bi: (ci, bi, 0)),
            pl.BlockSpec((NB, c), lambda p, ci, bi: (bi, ci)),
            pl.BlockSpec((1, 1, c), lambda p, ci, bi: (ci, 0, 0)),
            pl.BlockSpec((1, 1, c), lambda p, ci, bi: (ci, 0, 0)),
        ],
        out_specs=pl.BlockSpec((NB, c), lambda p, ci, bi: (bi, ci)),
        out_shape=jax.ShapeDtypeStruct((NP, dp), jnp.float32),
        scratch_shapes=[
            pltpu.VMEM((nc, 1, c), jnp.float32),
            pltpu.VMEM((nc, 1, c), jnp.float32),
        ],
    )(agg3, s2, g3, bt3)


def _pool(x5, batch3, dp):
    nb = NP // NB

    def body(x_ref, b_ref, out_ref, acc_ref):
        bi = pl.program_id(0)
        ids = b_ref[0]
        iot = lax.broadcasted_iota(jnp.int32, (NGROUPS, NB), 0)
        oh = (ids == iot).astype(jnp.float32)
        contrib = jnp.dot(oh, x_ref[...], preferred_element_type=jnp.float32,
                          precision=lax.Precision.HIGHEST)

        @pl.when(bi == 0)
        def _init():
            acc_ref[...] = contrib

        @pl.when(bi != 0)
        def _acc():
            acc_ref[...] += contrib

        @pl.when(bi == nb - 1)
        def _out():
            out_ref[...] = acc_ref[...]

    return pl.pallas_call(
        body,
        grid=(nb,),
        in_specs=[
            pl.BlockSpec((NB, dp), lambda bi: (bi, 0)),
            pl.BlockSpec((1, 1, NB), lambda bi: (bi, 0, 0)),
        ],
        out_specs=pl.BlockSpec((NGROUPS, dp), lambda bi: (0, 0)),
        out_shape=jax.ShapeDtypeStruct((NGROUPS, dp), jnp.float32),
        scratch_shapes=[pltpu.VMEM((NGROUPS, dp), jnp.float32)],
    )(x5, batch3)


def kernel(x, edge_index, batch, params):
    src = edge_index[0].astype(jnp.int32)
    dst = edge_index[1].astype(jnp.int32)
    e = src.shape[0]

    order = jnp.argsort(dst)
    src_s = src[order]
    dst_s = dst[order]
    rowptr = jnp.searchsorted(
        dst_s, jnp.arange(NP + 32, dtype=jnp.int32)).astype(jnp.int32)
    pad_e = jnp.zeros((2 * EB,), jnp.int32)
    src_p = jnp.concatenate([src_s, pad_e])
    dst_p = jnp.concatenate([dst_s, pad_e])

    x_cur = jnp.pad(x, ((0, NP - N_REAL), (0, 0)))
    batch_p = jnp.pad(batch.astype(jnp.int32), (0, NP - N_REAL),
                      constant_values=NGROUPS)
    batch3 = batch_p.reshape(NP // NB, 1, NB)

    dp_prev = x.shape[1]
    dp = dp_prev
    for (wk, wq, wv, ws, b, gamma, beta) in params:
        din, dout = wk.shape
        dp, c = _CHUNK[dout]
        nc = dp // c
        pad_w = ((0, dp_prev - din), (0, dp - dout))
        wk_p = jnp.pad(wk, pad_w)
        wq_p = jnp.pad(wq, pad_w)
        wv_p = jnp.pad(wv, pad_w)
        ws_p = jnp.pad(ws, pad_w)
        b3 = jnp.pad(b, (0, dp - dout)).reshape(nc, 1, c)
        g3 = jnp.pad(gamma, (0, dp - dout)).reshape(nc, 1, c)
        bt3 = jnp.pad(beta, (0, dp - dout)).reshape(nc, 1, c)

        ek3, eq3, v3, s2 = _dense(x_cur, wk_p, wq_p, wv_p, ws_p, b3, dp, c)
        agg3 = _edge(ek3, eq3, v3, rowptr, src_p, dst_p, dp, c)
        x_cur = _post(agg3, s2, g3, bt3, dp, c)
        dp_prev = dp

    return _pool(x_cur, batch3, dp)


# speedup vs baseline: 5.8381x; 1.0089x over previous
import functools

import jax
import jax.numpy as jnp
from jax import lax
from jax.experimental import pallas as pl
from jax.experimental.pallas import tpu as pltpu
from jax.experimental.pallas import tpu_sc as plsc

N_REAL = 10000
NP = 10240
NW = 32
ROWS = NP // NW
HROWS = ROWS // 2
EB = 128
NB = 512
NGROUPS = 64

_CHUNK = {1024: (1024, 128), 512: (512, 128), 256: (256, 128), 516: (640, 128)}


def _dense(x, wk, wq, wv, ws, b3, dp, c):
    nc = dp // c
    nb = NP // NB
    din = x.shape[1]

    def body(x_ref, wk_ref, wq_ref, wv_ref, ws_ref, b_ref,
             ek_ref, eq_ref, v_ref, s_ref):
        xb = x_ref[...]

        def mm(w_ref):
            return jnp.dot(xb, w_ref[...], preferred_element_type=jnp.float32)

        ek_ref[0] = jnp.exp(-mm(wk_ref))
        eq_ref[0] = jnp.exp(-mm(wq_ref))
        v_ref[0] = mm(wv_ref)
        s_ref[...] = mm(ws_ref) + b_ref[0]

    chunk3 = jax.ShapeDtypeStruct((nc, NP, c), jnp.float32)
    return pl.pallas_call(
        body,
        grid=(nc, nb),
        in_specs=[
            pl.BlockSpec((NB, din), lambda ci, bi: (bi, 0)),
            pl.BlockSpec((din, c), lambda ci, bi: (0, ci)),
            pl.BlockSpec((din, c), lambda ci, bi: (0, ci)),
            pl.BlockSpec((din, c), lambda ci, bi: (0, ci)),
            pl.BlockSpec((din, c), lambda ci, bi: (0, ci)),
            pl.BlockSpec((1, 1, c), lambda ci, bi: (ci, 0, 0)),
        ],
        out_specs=[
            pl.BlockSpec((1, NB, c), lambda ci, bi: (ci, bi, 0)),
            pl.BlockSpec((1, NB, c), lambda ci, bi: (ci, bi, 0)),
            pl.BlockSpec((1, NB, c), lambda ci, bi: (ci, bi, 0)),
            pl.BlockSpec((NB, c), lambda ci, bi: (bi, ci)),
        ],
        out_shape=[chunk3, chunk3, chunk3,
                   jax.ShapeDtypeStruct((NP, dp), jnp.float32)],
    )(x, wk, wq, wv, ws, b3)


def _edge(ek3, eq3, v3, rowptr, src_s, dst_s, dp, c):
    nc = dp // c
    ng = c // 16
    mesh = plsc.VectorSubcoreMesh(core_axis_name="c", subcore_axis_name="s",
                                  num_cores=2, num_subcores=16)

    @functools.partial(
        pl.kernel,
        out_type=jax.ShapeDtypeStruct((nc, NP, c), jnp.float32),
        mesh=mesh,
        scratch_types=[
            pltpu.VMEM((HROWS, c), jnp.float32),
            pltpu.VMEM((HROWS, c), jnp.float32),
            pltpu.VMEM((EB,), jnp.int32),
            pltpu.VMEM((EB,), jnp.int32),
            pltpu.VMEM((EB + 16,), jnp.int32),
            pltpu.VMEM((EB + 16,), jnp.int32),
            pltpu.VMEM((EB, c), jnp.float32),
            pltpu.VMEM((EB, c), jnp.float32),
            pltpu.VMEM((EB, c), jnp.float32),
            pltpu.VMEM((EB, c), jnp.float32),
            pltpu.VMEM((HROWS + 32,), jnp.int32),
            pltpu.SMEM((8,), jnp.int32),
            pltpu.SemaphoreType.DMA,
            pltpu.SemaphoreType.DMA,
            pltpu.SemaphoreType.DMA,
            pltpu.SemaphoreType.DMA,
            pltpu.SemaphoreType.DMA,
            pltpu.SemaphoreType.DMA,
            pltpu.SemaphoreType.DMA,
        ],
    )
    def k(ek_hbm, eq_hbm, v_hbm, rp_hbm, src_hbm, dst_hbm, agg_hbm,
          agg_v, ek_v, si0, si1, di0, di1, eq0, eq1, vv0, vv1, rp_v, rs_v,
          sa0, sa1, sb0, sb1, sx0, sx1, sek):
        sems = ((sa0, sb0), (sa1, sb1))
        isems = (sx0, sx1)
        sis = (si0, si1)
        dis = (di0, di1)
        eqs = (eq0, eq1)
        vvs = (vv0, vv1)
        wid = lax.axis_index("s") * 2 + lax.axis_index("c")
        sls = [pl.ds(g2 * 16, 16) for g2 in range(ng)]

        def rd(idx):
            return rp_v[pl.ds(idx, 16)][0]

        for ci in range(nc):
            for h in (0, 1):
                base = wid * ROWS + h * HROWS
                pltpu.sync_copy(rp_hbm.at[pl.ds(base, HROWS + 32)], rp_v)
                e_lo = rd(0)
                e_hi = rd(HROWS)
                e0 = (e_lo // 8) * 8
                nblk = (e_hi - e0 + EB - 1) // EB

                cp_ek = pltpu.async_copy(ek_hbm.at[ci, pl.ds(base, HROWS)],
                                         ek_v, sek)

                def issue_idx(g, buf):
                    bs = e0 + g * EB
                    pltpu.async_copy(src_hbm.at[pl.ds(bs, EB)], sis[buf],
                                     isems[buf])
                    pltpu.async_copy(dst_hbm.at[pl.ds(bs, EB + 16)],
                                     dis[buf], isems[buf])

                def issue_gather(g, buf):
                    bs = e0 + g * EB
                    pltpu.make_async_copy(src_hbm.at[pl.ds(bs, EB)],
                                          sis[buf], isems[buf]).wait()
                    pltpu.make_async_copy(dst_hbm.at[pl.ds(bs, EB + 16)],
                                          dis[buf], isems[buf]).wait()
                    pltpu.async_copy(eq_hbm.at[ci].at[sis[buf]],
                                     eqs[buf], sems[buf][0])
                    pltpu.async_copy(v_hbm.at[ci].at[sis[buf]],
                                     vvs[buf], sems[buf][1])

                def compute(g, buf):
                    bs = e0 + g * EB
                    pltpu.make_async_copy(eq_hbm.at[ci].at[sis[buf]],
                                          eqs[buf], sems[buf][0]).wait()
                    pltpu.make_async_copy(v_hbm.at[ci].at[sis[buf]],
                                          vvs[buf], sems[buf][1]).wait()
                    lo = jnp.maximum(e_lo, bs) - bs
                    hi = jnp.minimum(e_hi, bs + EB) - bs
                    rstart = rs_v[0]
                    jsafe = jnp.maximum(hi - 1, 0)
                    rlast = dis[buf][pl.ds(jsafe, 16)][0] - base
                    nrows = jnp.where(hi > lo, rlast - rstart + 1, 0)

                    def row(ri, jlo):
                        r = rstart + ri
                        rend = rd(r + 1) - bs
                        jhi = jnp.maximum(jnp.minimum(rend, hi), jlo)
                        ekl = [ek_v[r, sl] for sl in sls]

                        def erun(jj, accs):
                            eql = [eqs[buf][jj, sl] for sl in sls]
                            vvl = [vvs[buf][jj, sl] for sl in sls]
                            tl = [a * b for a, b in zip(ekl, eql)]
                            dl = [t + 1.0 for t in tl]
                            cl = [v / d for v, d in zip(vvl, dl)]
                            return tuple(a + cv for a, cv in
                                         zip(accs, cl))

                        accs = lax.fori_loop(
                            jlo, jhi, erun,
                            tuple(jnp.zeros((16,), jnp.float32)
                                  for _ in range(ng)))

                        for sl, av in zip(sls, accs):
                            plsc.addupdate(agg_v.at[r, sl], av)

                        return jhi

                    lax.fori_loop(0, nrows, row, lo)
                    rs_v[0] = jnp.where(hi > lo, rlast, rstart)

                rs_v[0] = 0

                @pl.when(nblk > 0)
                def _prime0():
                    issue_idx(0, 0)

                @pl.when(nblk > 1)
                def _prime1():
                    issue_idx(1, 1)

                def zrow(r, carry):
                    for g in range(ng):
                        agg_v[r, pl.ds(g * 16, 16)] = jnp.zeros((16,),
                                                                jnp.float32)
                    return carry
                lax.fori_loop(0, HROWS, zrow, 0)
                cp_ek.wait()

                @pl.when(nblk > 0)
                def _prime2():
                    issue_gather(0, 0)

                def pair(ip, carry):
                    for b in (0, 1):
                        g = ip * 2 + b

                        @pl.when(g < nblk)
                        def _step():
                            @pl.when(g + 1 < nblk)
                            def _ahead():
                                issue_gather(g + 1, 1 - b)
                            compute(g, b)

                            @pl.when(g + 2 < nblk)
                            def _stage():
                                issue_idx(g + 2, b)
                    return carry

                lax.fori_loop(0, (nblk + 1) // 2, pair, 0)
                pltpu.sync_copy(agg_v, agg_hbm.at[ci, pl.ds(base, HROWS)])

    return k(ek3, eq3, v3, rowptr, src_s, dst_s)


def _post(agg3, s2, g3, bt3, dp, c):
    nc = dp // c
    nb = NP // NB
    inv_n = 1.0 / N_REAL

    def body(agg_ref, s_ref, g_ref, bt_ref, out_ref, sum_ref, ssq_ref):
        p = pl.program_id(0)
        ci = pl.program_id(1)
        bi = pl.program_id(2)
        row = bi * NB + lax.broadcasted_iota(jnp.int32, (NB, 1), 0)
        live = row < N_REAL
        y = jnp.where(live, jnp.maximum(agg_ref[0] + s_ref[...], 0.0), 0.0)

        @pl.when(p == 0)
        def _stats():
            cs = jnp.sum(y, axis=0, keepdims=True)
            cq = jnp.sum(y * y, axis=0, keepdims=True)

            @pl.when(bi == 0)
            def _init():
                sum_ref[ci] = cs
                ssq_ref[ci] = cq

            @pl.when(bi != 0)
            def _acc():
                sum_ref[ci] += cs
                ssq_ref[ci] += cq

        @pl.when(p == 1)
        def _norm():
            mu = sum_ref[ci] * inv_n
            var = ssq_ref[ci] * inv_n - mu * mu
            rs = lax.rsqrt(var + 1e-5)
            out_ref[...] = jnp.where(
                live, (y - mu) * (rs * g_ref[0]) + bt_ref[0], 0.0)

    return pl.pallas_call(
        body,
        grid=(2, nc, nb),
        in_specs=[
            pl.BlockSpec((1, NB, c), lambda p, ci, bi: (ci, bi, 0)),
            pl.BlockSpec((NB, c), lambda p, ci, bi: (bi, ci)),
            pl.BlockSpec((1, 1, c), lambda p, ci, bi: (ci, 0, 0)),
            pl.BlockSpec((1, 1, c), lambda p, ci, bi: (ci, 0, 0)),
        ],
        out_specs=pl.BlockSpec((NB, c), lambda p, ci, bi: (bi, ci)),
        out_shape=jax.ShapeDtypeStruct((NP, dp), jnp.float32),
        scratch_shapes=[
            pltpu.VMEM((nc, 1, c), jnp.float32),
            pltpu.VMEM((nc, 1, c), jnp.float32),
        ],
    )(agg3, s2, g3, bt3)


def _pool(x5, batch3, dp):
    nb = NP // NB

    def body(x_ref, b_ref, out_ref, acc_ref):
        bi = pl.program_id(0)
        ids = b_ref[0]
        iot = lax.broadcasted_iota(jnp.int32, (NGROUPS, NB), 0)
        oh = (ids == iot).astype(jnp.float32)
        contrib = jnp.dot(oh, x_ref[...], preferred_element_type=jnp.float32,
                          precision=lax.Precision.HIGHEST)

        @pl.when(bi == 0)
        def _init():
            acc_ref[...] = contrib

        @pl.when(bi != 0)
        def _acc():
            acc_ref[...] += contrib

        @pl.when(bi == nb - 1)
        def _out():
            out_ref[...] = acc_ref[...]

    return pl.pallas_call(
        body,
        grid=(nb,),
        in_specs=[
            pl.BlockSpec((NB, dp), lambda bi: (bi, 0)),
            pl.BlockSpec((1, 1, NB), lambda bi: (bi, 0, 0)),
        ],
        out_specs=pl.BlockSpec((NGROUPS, dp), lambda bi: (0, 0)),
        out_shape=jax.ShapeDtypeStruct((NGROUPS, dp), jnp.float32),
        scratch_shapes=[pltpu.VMEM((NGROUPS, dp), jnp.float32)],
    )(x5, batch3)


def kernel(x, edge_index, batch, params):
    src = edge_index[0].astype(jnp.int32)
    dst = edge_index[1].astype(jnp.int32)
    e = src.shape[0]

    order = jnp.argsort(dst)
    src_s = src[order]
    dst_s = dst[order]
    rowptr = jnp.searchsorted(
        dst_s, jnp.arange(NP + 32, dtype=jnp.int32)).astype(jnp.int32)
    pad_e = jnp.zeros((2 * EB,), jnp.int32)
    src_p = jnp.concatenate([src_s, pad_e])
    dst_p = jnp.concatenate([dst_s, pad_e])

    x_cur = jnp.pad(x, ((0, NP - N_REAL), (0, 0)))
    batch_p = jnp.pad(batch.astype(jnp.int32), (0, NP - N_REAL),
                      constant_values=NGROUPS)
    batch3 = batch_p.reshape(NP // NB, 1, NB)

    dp_prev = x.shape[1]
    dp = dp_prev
    for (wk, wq, wv, ws, b, gamma, beta) in params:
        din, dout = wk.shape
        dp, c = _CHUNK[dout]
        nc = dp // c
        pad_w = ((0, dp_prev - din), (0, dp - dout))
        wk_p = jnp.pad(wk, pad_w)
        wq_p = jnp.pad(wq, pad_w)
        wv_p = jnp.pad(wv, pad_w)
        ws_p = jnp.pad(ws, pad_w)
        b3 = jnp.pad(b, (0, dp - dout)).reshape(nc, 1, c)
        g3 = jnp.pad(gamma, (0, dp - dout)).reshape(nc, 1, c)
        bt3 = jnp.pad(beta, (0, dp - dout)).reshape(nc, 1, c)

        ek3, eq3, v3, s2 = _dense(x_cur, wk_p, wq_p, wv_p, ws_p, b3, dp, c)
        agg3 = _edge(ek3, eq3, v3, rowptr, src_p, dst_p, dp, c)
        x_cur = _post(agg3, s2, g3, bt3, dp, c)
        dp_prev = dp

    return _pool(x_cur, batch3, dp)
